# baseline (speedup 1.0000x reference)
import functools

import jax
import jax.numpy as jnp
from jax import lax
from jax.experimental import pallas as pl
from jax.experimental.pallas import tpu as pltpu
from jax.experimental.pallas import tpu_sc as plsc

L = 16

_GDN = lax.GatherDimensionNumbers(
    offset_dims=(), collapsed_slice_dims=(0,), start_index_map=(0,))


def _lane_shuffle(x, perm):
    return lax.gather(x, perm[:, None], _GDN, slice_sizes=(1,),
                      mode=lax.GatherScatterMode.PROMISE_IN_BOUNDS)


def _butterfly(x, op):
    lanes = lax.iota(jnp.int32, L)
    for k in (1, 2, 4, 8):
        x = op(x, _lane_shuffle(x, lax.bitwise_xor(lanes, k)))
    return x


def _worker_id():
    nc = plsc.get_sparse_core_info().num_cores
    return lax.axis_index("s") * nc + lax.axis_index("c")


def _seg_sum(rv, r0, S):
    a0 = jnp.zeros((L,), jnp.float32)
    a1 = jnp.zeros((L,), jnp.float32)
    for s in range(S):
        a0 = a0 + rv[r0 + s, pl.ds(0, L)]
        a1 = a1 + rv[r0 + s, pl.ds(L, L)]
    return a0, a1


def _gather_body(nch, G, S, E, ctx_hbm, tab, sums_hbm,
                 idx_all, rows0, rows1, acc0, acc1, sem0, sem1, semo0, semo1):
    wid = _worker_id()
    rows = (rows0, rows1)
    sems = (sem0, sem1)
    accs = (acc0, acc1)
    semos = (semo0, semo1)

    pltpu.sync_copy(ctx_hbm.at[wid], idx_all)
    pltpu.async_copy(tab.at[idx_all.at[0]], rows0, sem0)

    def do_chunk(c, par):
        pltpu.make_async_copy(tab.at[idx_all.at[c]], rows[par], sems[par]).wait()

        @pl.when(c + 1 < nch)
        def _():
            pltpu.async_copy(tab.at[idx_all.at[c + 1]], rows[1 - par],
                             sems[1 - par])

        @pl.when(c >= 2)
        def _():
            pltpu.make_async_copy(
                accs[par], sums_hbm.at[wid, c], semos[par]).wait()

        rv = rows[par]
        acc = accs[par]

        def seg_body(g, carry):
            a0, a1 = _seg_sum(rv, g * S, S)
            acc[g, pl.ds(0, L)] = a0
            acc[g, pl.ds(L, L)] = a1
            return carry

        lax.fori_loop(0, G, seg_body, 0)
        pltpu.async_copy(acc, sums_hbm.at[wid, c], semos[par])

    def pair_body(c2, carry):
        do_chunk(2 * c2, 0)
        do_chunk(2 * c2 + 1, 1)
        return carry

    lax.fori_loop(0, nch // 2, pair_body, 0)
    for par in range(2):
        pltpu.make_async_copy(accs[par], sums_hbm.at[wid, 0], semos[par]).wait()


def _final_body(nch, G, S, E, ctx_hbm, tab2, tab3, s1_hbm, out_hbm,
                idx_all, rows0, rows1, s1b0, s1b1, outc, ov0, ov1,
                sem0, sem1, sems0, sems1, semo0, semo1):
    wid = _worker_id()
    brow0 = wid * nch
    rows = (rows0, rows1)
    sems = (sem0, sem1)
    s1bs = (s1b0, s1b1)
    sems_s = (sems0, sems1)
    ovs = (ov0, ov1)
    semos = (semo0, semo1)
    inv_e = 1.0 / E

    pltpu.sync_copy(ctx_hbm.at[wid], idx_all)
    pltpu.async_copy(tab2.at[idx_all.at[0]], rows0, sem0)
    pltpu.async_copy(s1_hbm.at[wid, 0], s1b0, sems0)

    def do_chunk(c, half):
        pltpu.make_async_copy(tab2.at[idx_all.at[c]], rows0, sem0).wait()
        pltpu.async_copy(tab3.at[idx_all.at[c]], rows1, sem1)

        def sum_body(g, carry):
            a0, a1 = _seg_sum(rows0, g * S, S)
            outc[g, pl.ds(0, L)] = a0
            outc[g, pl.ds(L, L)] = a1
            return carry

        lax.fori_loop(0, G, sum_body, 0)

        pltpu.make_async_copy(tab3.at[idx_all.at[c]], rows1, sem1).wait()
        pltpu.make_async_copy(s1_hbm.at[wid, c], s1bs[half], sems_s[half]).wait()

        @pl.when(c + 1 < nch)
        def _():
            pltpu.async_copy(tab2.at[idx_all.at[c + 1]], rows0, sem0)
            pltpu.async_copy(s1_hbm.at[wid, c + 1], s1bs[1 - half],
                             sems_s[1 - half])

        @pl.when(c >= 2)
        def _():
            pltpu.make_async_copy(
                ovs[half], out_hbm.at[brow0 + c], semos[half]).wait()

        s1b = s1bs[half]
        ov = ovs[half]

        def seg_body(g, carry):
            a0, a1 = _seg_sum(rows1, g * S, S)
            s1a = s1b[g, pl.ds(0, L)]
            s1c = s1b[g, pl.ds(L, L)]
            s2a = outc[g, pl.ds(0, L)]
            s2c = outc[g, pl.ds(L, L)]
            qa = s1a * inv_e
            qb = s1c * inv_e
            za = s1a * qa
            zb = s1c * qb
            m = _butterfly(jnp.maximum(za, zb), jnp.maximum)
            ea = jnp.exp(za - m)
            eb = jnp.exp(zb - m)
            r = 1.0 / _butterfly(ea + eb, jnp.add)
            qa = qa + s2a * ea * r
            qb = qb + s2c * eb * r
            za = s2a * qa
            zb = s2c * qb
            m = _butterfly(jnp.maximum(za, zb), jnp.maximum)
            ea = jnp.exp(za - m)
            eb = jnp.exp(zb - m)
            r = 1.0 / _butterfly(ea + eb, jnp.add)
            ov[g, pl.ds(0, L)] = a0 * ea * r
            ov[g, pl.ds(L, L)] = a1 * eb * r
            return carry

        lax.fori_loop(0, G, seg_body, 0)
        pltpu.async_copy(ov, out_hbm.at[brow0 + c], semos[half])

    def pair_body(c2, carry):
        do_chunk(2 * c2, 0)
        do_chunk(2 * c2 + 1, 1)
        return carry

    lax.fori_loop(0, nch // 2, pair_body, 0)
    for half in range(2):
        pltpu.make_async_copy(ovs[half], out_hbm.at[brow0], semos[half]).wait()


def _sc_encoder(ctx, t1, t2, t3, B, M, S, E):
    info = plsc.get_sparse_core_info()
    nworkers = info.num_cores * info.num_subcores
    seg_per_worker = (B * M) // nworkers
    G = M
    nch = seg_per_worker // G
    assert seg_per_worker % G == 0 and nch % 2 == 0 and (G * S) % 8 == 0

    ctx3 = ctx.reshape(nworkers, nch, G * S)
    mesh = plsc.VectorSubcoreMesh(core_axis_name="c", subcore_axis_name="s")
    params = pltpu.CompilerParams(use_tc_tiling_on_sc=False)

    gather_fn = pl.kernel(
        functools.partial(_gather_body, nch, G, S, E),
        out_type=jax.ShapeDtypeStruct((nworkers, nch, G, E), jnp.float32),
        mesh=mesh,
        scratch_types=[
            pltpu.VMEM((nch, G * S), jnp.int32),
            pltpu.VMEM((G * S, E), jnp.float32),
            pltpu.VMEM((G * S, E), jnp.float32),
            pltpu.VMEM((G, E), jnp.float32),
            pltpu.VMEM((G, E), jnp.float32),
            pltpu.SemaphoreType.DMA,
            pltpu.SemaphoreType.DMA,
            pltpu.SemaphoreType.DMA,
            pltpu.SemaphoreType.DMA,
        ],
        compiler_params=params,
    )
    sums1 = gather_fn(ctx3, t1)

    final_fn = pl.kernel(
        functools.partial(_final_body, nch, G, S, E),
        out_type=jax.ShapeDtypeStruct((B, M, E), jnp.float32),
        mesh=mesh,
        scratch_types=[
            pltpu.VMEM((nch, G * S), jnp.int32),
            pltpu.VMEM((G * S, E), jnp.float32),
            pltpu.VMEM((G * S, E), jnp.float32),
            pltpu.VMEM((G, E), jnp.float32),
            pltpu.VMEM((G, E), jnp.float32),
            pltpu.VMEM((G, E), jnp.float32),
            pltpu.VMEM((G, E), jnp.float32),
            pltpu.VMEM((G, E), jnp.float32),
            pltpu.SemaphoreType.DMA,
            pltpu.SemaphoreType.DMA,
            pltpu.SemaphoreType.DMA,
            pltpu.SemaphoreType.DMA,
            pltpu.SemaphoreType.DMA,
            pltpu.SemaphoreType.DMA,
        ],
        compiler_params=params,
    )
    return final_fn(ctx3, t2, t3, sums1)


def kernel(context, A_tables, C_last):
    B, M, S = context.shape
    E = A_tables.shape[-1]
    ctx = context.reshape(-1)
    return _sc_encoder(ctx, A_tables[1], A_tables[2], C_last, B, M, S, E)


# speedup vs baseline: 1.0070x; 1.0070x over previous
import functools

import jax
import jax.numpy as jnp
from jax import lax
from jax.experimental import pallas as pl
from jax.experimental.pallas import tpu as pltpu
from jax.experimental.pallas import tpu_sc as plsc

L = 16

_GDN = lax.GatherDimensionNumbers(
    offset_dims=(), collapsed_slice_dims=(0,), start_index_map=(0,))


def _lane_shuffle(x, perm):
    return lax.gather(x, perm[:, None], _GDN, slice_sizes=(1,),
                      mode=lax.GatherScatterMode.PROMISE_IN_BOUNDS)


def _butterfly(x, op):
    lanes = lax.iota(jnp.int32, L)
    for k in (1, 2, 4, 8):
        x = op(x, _lane_shuffle(x, lax.bitwise_xor(lanes, k)))
    return x


def _worker_id():
    nc = plsc.get_sparse_core_info().num_cores
    return lax.axis_index("s") * nc + lax.axis_index("c")


def _seg_sum(rv, r0, S):
    a0 = jnp.zeros((L,), jnp.float32)
    a1 = jnp.zeros((L,), jnp.float32)
    for s in range(S):
        a0 = a0 + rv[r0 + s, pl.ds(0, L)]
        a1 = a1 + rv[r0 + s, pl.ds(L, L)]
    return a0, a1


def _stage_idx(ctx_hbm, idx_all, wid, nch):
    pltpu.sync_copy(ctx_hbm.at[wid], idx_all)


def _chunk_idx(idx_all, c):
    return idx_all.at[c]


def _gather_body(nch, G, S, E, ctx_hbm, tab, sums_hbm,
                 idx_all, rows0, rows1, acc0, acc1, sem0, sem1, semo0, semo1):
    wid = _worker_id()
    rows = (rows0, rows1)
    sems = (sem0, sem1)
    accs = (acc0, acc1)
    semos = (semo0, semo1)

    _stage_idx(ctx_hbm, idx_all, wid, nch)
    pltpu.async_copy(tab.at[_chunk_idx(idx_all, 0)], rows0, sem0)

    def do_chunk(c, par):
        pltpu.make_async_copy(tab.at[_chunk_idx(idx_all, c)], rows[par], sems[par]).wait()

        @pl.when(c + 1 < nch)
        def _():
            pltpu.async_copy(tab.at[_chunk_idx(idx_all, c + 1)], rows[1 - par],
                             sems[1 - par])

        @pl.when(c >= 2)
        def _():
            pltpu.make_async_copy(
                accs[par], sums_hbm.at[wid, c], semos[par]).wait()

        rv = rows[par]
        acc = accs[par]

        def seg_body(g, carry):
            a0, a1 = _seg_sum(rv, g * S, S)
            acc[g, pl.ds(0, L)] = a0
            acc[g, pl.ds(L, L)] = a1
            return carry

        lax.fori_loop(0, G, seg_body, 0)
        pltpu.async_copy(acc, sums_hbm.at[wid, c], semos[par])

    def pair_body(c2, carry):
        do_chunk(2 * c2, 0)
        do_chunk(2 * c2 + 1, 1)
        return carry

    lax.fori_loop(0, nch // 2, pair_body, 0)
    for par in range(2):
        pltpu.make_async_copy(accs[par], sums_hbm.at[wid, 0], semos[par]).wait()


def _final_body(nch, G, S, E, ctx_hbm, tab, s1_hbm, s2_hbm, out_hbm,
                idx_all, rows0, rows1, s1b0, s1b1, s2b0, s2b1, ov0, ov1,
                sem0, sem1, sems0, sems1, semo0, semo1):
    wid = _worker_id()
    brow0 = wid * nch
    rows = (rows0, rows1)
    sems = (sem0, sem1)
    s1bs = (s1b0, s1b1)
    s2bs = (s2b0, s2b1)
    sems_s = (sems0, sems1)
    ovs = (ov0, ov1)
    semos = (semo0, semo1)
    inv_e = 1.0 / E

    _stage_idx(ctx_hbm, idx_all, wid, nch)
    pltpu.async_copy(tab.at[_chunk_idx(idx_all, 0)], rows0, sem0)
    pltpu.async_copy(s1_hbm.at[wid, 0], s1b0, sems0)
    pltpu.async_copy(s2_hbm.at[wid, 0], s2b0, sems0)

    def do_chunk(c, par):
        pltpu.make_async_copy(tab.at[_chunk_idx(idx_all, c)], rows[par], sems[par]).wait()
        pltpu.make_async_copy(s1_hbm.at[wid, c], s1bs[par], sems_s[par]).wait()
        pltpu.make_async_copy(s2_hbm.at[wid, c], s2bs[par], sems_s[par]).wait()

        @pl.when(c + 1 < nch)
        def _():
            pltpu.async_copy(tab.at[_chunk_idx(idx_all, c + 1)], rows[1 - par],
                             sems[1 - par])
            pltpu.async_copy(s1_hbm.at[wid, c + 1], s1bs[1 - par],
                             sems_s[1 - par])
            pltpu.async_copy(s2_hbm.at[wid, c + 1], s2bs[1 - par],
                             sems_s[1 - par])

        @pl.when(c >= 2)
        def _():
            pltpu.make_async_copy(
                ovs[par], out_hbm.at[brow0 + c], semos[par]).wait()

        rv = rows[par]
        s1b = s1bs[par]
        s2b = s2bs[par]
        ov = ovs[par]

        def seg_body(g, carry):
            a0, a1 = _seg_sum(rv, g * S, S)
            s1a = s1b[g, pl.ds(0, L)]
            s1c = s1b[g, pl.ds(L, L)]
            s2a = s2b[g, pl.ds(0, L)]
            s2c = s2b[g, pl.ds(L, L)]
            qa = s1a * inv_e
            qb = s1c * inv_e
            za = s1a * qa
            zb = s1c * qb
            m = _butterfly(jnp.maximum(za, zb), jnp.maximum)
            ea = jnp.exp(za - m)
            eb = jnp.exp(zb - m)
            r = 1.0 / _butterfly(ea + eb, jnp.add)
            qa = qa + s2a * ea * r
            qb = qb + s2c * eb * r
            za = s2a * qa
            zb = s2c * qb
            m = _butterfly(jnp.maximum(za, zb), jnp.maximum)
            ea = jnp.exp(za - m)
            eb = jnp.exp(zb - m)
            r = 1.0 / _butterfly(ea + eb, jnp.add)
            ov[g, pl.ds(0, L)] = a0 * ea * r
            ov[g, pl.ds(L, L)] = a1 * eb * r
            return carry

        lax.fori_loop(0, G, seg_body, 0)
        pltpu.async_copy(ov, out_hbm.at[brow0 + c], semos[par])

    def pair_body(c2, carry):
        do_chunk(2 * c2, 0)
        do_chunk(2 * c2 + 1, 1)
        return carry

    lax.fori_loop(0, nch // 2, pair_body, 0)
    for par in range(2):
        pltpu.make_async_copy(ovs[par], out_hbm.at[brow0], semos[par]).wait()


def _sc_encoder(context, t1, t2, t3, B, M, S, E):
    info = plsc.get_sparse_core_info()
    nworkers = info.num_cores * info.num_subcores
    seg_per_worker = (B * M) // nworkers
    G = M
    nch = seg_per_worker // G
    assert seg_per_worker % G == 0 and nch % 2 == 0 and (G * S) % 8 == 0

    mesh = plsc.VectorSubcoreMesh(core_axis_name="c", subcore_axis_name="s")
    params = pltpu.CompilerParams(use_tc_tiling_on_sc=False)

    gather_fn = pl.kernel(
        functools.partial(_gather_body, nch, G, S, E),
        out_type=jax.ShapeDtypeStruct((nworkers, nch, G, E), jnp.float32),
        mesh=mesh,
        scratch_types=[
            pltpu.VMEM((nch, G * S), jnp.int32),
            pltpu.VMEM((G * S, E), jnp.float32),
            pltpu.VMEM((G * S, E), jnp.float32),
            pltpu.VMEM((G, E), jnp.float32),
            pltpu.VMEM((G, E), jnp.float32),
            pltpu.SemaphoreType.DMA,
            pltpu.SemaphoreType.DMA,
            pltpu.SemaphoreType.DMA,
            pltpu.SemaphoreType.DMA,
        ],
        compiler_params=params,
    )
    ctx3 = context.reshape(nworkers, nch, G * S)
    sums1 = gather_fn(ctx3, t1)
    sums2 = gather_fn(ctx3, t2)

    final_fn = pl.kernel(
        functools.partial(_final_body, nch, G, S, E),
        out_type=jax.ShapeDtypeStruct((B, M, E), jnp.float32),
        mesh=mesh,
        scratch_types=[
            pltpu.VMEM((nch, G * S), jnp.int32),
            pltpu.VMEM((G * S, E), jnp.float32),
            pltpu.VMEM((G * S, E), jnp.float32),
            pltpu.VMEM((G, E), jnp.float32),
            pltpu.VMEM((G, E), jnp.float32),
            pltpu.VMEM((G, E), jnp.float32),
            pltpu.VMEM((G, E), jnp.float32),
            pltpu.VMEM((G, E), jnp.float32),
            pltpu.VMEM((G, E), jnp.float32),
            pltpu.SemaphoreType.DMA,
            pltpu.SemaphoreType.DMA,
            pltpu.SemaphoreType.DMA,
            pltpu.SemaphoreType.DMA,
            pltpu.SemaphoreType.DMA,
            pltpu.SemaphoreType.DMA,
        ],
        compiler_params=params,
    )
    return final_fn(ctx3, t3, sums1, sums2)


def kernel(context, A_tables, C_last):
    B, M, S = context.shape
    E = A_tables.shape[-1]
    return _sc_encoder(context, A_tables[1], A_tables[2], C_last, B, M, S, E)
